# Initial kernel scaffold; baseline (speedup 1.0000x reference)
import jax
import jax.numpy as jnp
from jax.experimental import pallas as pl


def kernel(uid, iid, user_table, uW1, ub1, uW2, ub2, item_table, iW1, ib1, iW2, ib2):
    raise NotImplementedError("write your pallas kernel here")


# speedup vs baseline: 1.0482x; 1.0482x over previous
import functools

import jax
import jax.numpy as jnp
from jax import lax
from jax.experimental import pallas as pl
from jax.experimental.pallas import tpu as pltpu
from jax.experimental.pallas import tpu_sc as plsc

B = 16384
D = 64
H1 = 128
H2 = 64

NC = 2
NS = 16
NW = NC * NS
B_PER_W = B // NW


def _sc_gather(uid, iid, user_table, item_table):
    mesh = plsc.VectorSubcoreMesh(core_axis_name="c", subcore_axis_name="s")

    @functools.partial(
        pl.kernel,
        mesh=mesh,
        compiler_params=pltpu.CompilerParams(use_tc_tiling_on_sc=False),
        out_type=[
            jax.ShapeDtypeStruct((B, D), jnp.float32),
            jax.ShapeDtypeStruct((B, D), jnp.float32),
        ],
        scratch_types=[
            pltpu.VMEM((B_PER_W,), jnp.int32),
            pltpu.VMEM((B_PER_W,), jnp.int32),
            pltpu.VMEM((B_PER_W, D), jnp.float32),
            pltpu.VMEM((B_PER_W, D), jnp.float32),
            pltpu.SemaphoreType.DMA,
            pltpu.SemaphoreType.DMA,
        ],
    )
    def k(uid_hbm, iid_hbm, ut_hbm, it_hbm, uout_hbm, iout_hbm,
          uidx_v, iidx_v, urows_v, irows_v, sem_u, sem_i):
        wid = lax.axis_index("s") * NC + lax.axis_index("c")
        base = wid * B_PER_W
        pltpu.sync_copy(uid_hbm.at[pl.ds(base, B_PER_W)], uidx_v)
        pltpu.sync_copy(iid_hbm.at[pl.ds(base, B_PER_W)], iidx_v)
        cu = pltpu.async_copy(ut_hbm.at[uidx_v], urows_v, sem_u)
        ci = pltpu.async_copy(it_hbm.at[iidx_v], irows_v, sem_i)
        cu.wait()
        pltpu.sync_copy(urows_v, uout_hbm.at[pl.ds(base, B_PER_W)])
        ci.wait()
        pltpu.sync_copy(irows_v, iout_hbm.at[pl.ds(base, B_PER_W)])

    return k(uid, iid, user_table, item_table)


def _mlp_body(urows, irows, uw1, ub1, uw2, ub2, iw1, ib1, iw2, ib2, out):
    u = jnp.dot(urows[:], uw1[:], preferred_element_type=jnp.float32) + ub1[:]
    u = jnp.maximum(u, 0.0)
    u = jnp.dot(u, uw2[:], preferred_element_type=jnp.float32) + ub2[:]
    u = jnp.maximum(u, 0.0)
    v = jnp.dot(irows[:], iw1[:], preferred_element_type=jnp.float32) + ib1[:]
    v = jnp.maximum(v, 0.0)
    v = jnp.dot(v, iw2[:], preferred_element_type=jnp.float32) + ib2[:]
    v = jnp.maximum(v, 0.0)
    out[:] = jnp.maximum(jnp.sum(u * v, axis=1), 0.0).reshape(out.shape)


BLK = 1024


def _tc_mlp(urows, irows, uW1, ub1, uW2, ub2, iW1, ib1, iW2, ib2):
    nblk = B // BLK
    row_spec = pl.BlockSpec((BLK, D), lambda i: (i, 0))
    w1_spec = pl.BlockSpec((D, H1), lambda i: (0, 0))
    b1_spec = pl.BlockSpec((1, H1), lambda i: (0, 0))
    w2_spec = pl.BlockSpec((H1, H2), lambda i: (0, 0))
    b2_spec = pl.BlockSpec((1, H2), lambda i: (0, 0))
    out = pl.pallas_call(
        _mlp_body,
        grid=(nblk,),
        in_specs=[row_spec, row_spec,
                  w1_spec, b1_spec, w2_spec, b2_spec,
                  w1_spec, b1_spec, w2_spec, b2_spec],
        out_specs=pl.BlockSpec((BLK // 128, 128), lambda i: (i, 0)),
        out_shape=jax.ShapeDtypeStruct((B // 128, 128), jnp.float32),
    )(urows, irows,
      uW1, ub1.reshape(1, H1), uW2, ub2.reshape(1, H2),
      iW1, ib1.reshape(1, H1), iW2, ib2.reshape(1, H2))
    return out.reshape(-1)


def kernel(uid, iid, user_table, uW1, ub1, uW2, ub2, item_table, iW1, ib1, iW2, ib2):
    uid = uid.astype(jnp.int32)
    iid = iid.astype(jnp.int32)
    urows, irows = _sc_gather(uid, iid, user_table, item_table)
    return _tc_mlp(urows, irows, uW1, ub1, uW2, ub2, iW1, ib1, iW2, ib2)
